# Initial kernel scaffold; baseline (speedup 1.0000x reference)
import jax
import jax.numpy as jnp
from jax.experimental import pallas as pl


def kernel(input, target, W):
    raise NotImplementedError("write your pallas kernel here")


# speedup vs baseline: 34.2214x; 34.2214x over previous
import functools

import jax
import jax.numpy as jnp
from jax import lax
from jax.experimental import pallas as pl
from jax.experimental.pallas import tpu as pltpu
from jax.experimental.pallas import tpu_sc as plsc

FEAT = 64
B = 1024
C_DIM = 100000
NW = 32
ROWS_W = 16
COLS_W = 128

COEF = 1.0 / (1.0 + max(5.0, 1500.0 / 1.1))

T1 = 0.7071067818211393
T2 = 1.7948965149208059e-09
T3 = -0.7071067792827723
T4 = -1.0


def _sc_gather_body(w_hbm, tgt_hbm, out_hbm, tgt_v, idx_v, rows_v, sem):
    wid = lax.axis_index("s") * 2 + lax.axis_index("c")

    pltpu.sync_copy(tgt_hbm, tgt_v)

    def build(c, carry):
        f = 2 * wid + c // 64
        val = tgt_v[pl.ds((c % 64) * 16, 16)] + f * C_DIM
        idx_v[c // 8, pl.ds((c % 8) * 16, 16)] = val
        return carry

    lax.fori_loop(0, 128, build, 0)

    def fire(j, carry):
        pltpu.async_copy(w_hbm.at[idx_v.at[j]], rows_v.at[j], sem)
        return carry

    lax.fori_loop(0, ROWS_W, fire, 0)

    def drain(j, carry):
        pltpu.make_async_copy(w_hbm.at[idx_v.at[j]], rows_v.at[j], sem).wait()
        return carry

    lax.fori_loop(0, ROWS_W, drain, 0)

    pltpu.sync_copy(rows_v, out_hbm.at[wid])


@functools.cache
def _sc_gather():
    return pl.kernel(
        _sc_gather_body,
        out_type=jax.ShapeDtypeStruct((NW, ROWS_W, COLS_W), jnp.float32),
        mesh=plsc.VectorSubcoreMesh(core_axis_name="c", subcore_axis_name="s"),
        scratch_types=[
            pltpu.VMEM((B,), jnp.int32),
            pltpu.VMEM((ROWS_W, COLS_W), jnp.int32),
            pltpu.VMEM((ROWS_W, COLS_W), jnp.float32),
            pltpu.SemaphoreType.DMA,
        ],
    )


def _tc_loss_body(x_ref, wt_ref, tcol_ref, trow_ref, out_ref):
    x = x_ref[...]
    wt = wt_ref[...]
    tcol = tcol_ref[...]
    trow = trow_ref[...]

    wnorm = jnp.sqrt(jnp.sum(wt * wt, axis=0, keepdims=True))
    scale = jnp.where(wnorm > 1e-5, 1e-5 / jnp.maximum(wnorm, 1e-30), 1.0) * 1e5
    ww = wt * scale
    wlen = jnp.sqrt(jnp.sum(ww * ww, axis=0, keepdims=True))
    xlen = jnp.sqrt(jnp.sum(x * x, axis=1, keepdims=True))

    dot = lax.dot_general(
        x, ww, (((1,), (0,)), ((), ())),
        precision=lax.Precision.HIGHEST,
        preferred_element_type=jnp.float32,
    )

    cos = jnp.clip(dot / xlen / wlen, -1.0, 1.0)
    cos2 = cos * cos
    cos_m = 8.0 * cos2 * cos2 - 8.0 * cos2 + 1.0
    k = ((cos <= T1).astype(jnp.float32)
         + (cos <= T2).astype(jnp.float32)
         + (cos <= T3).astype(jnp.float32)
         + (cos <= T4).astype(jnp.float32))
    sign = jnp.where(jnp.mod(k, 2.0) == 0.0, 1.0, -1.0)
    phi = sign * cos_m - 2.0 * k

    mask = (tcol == trow).astype(jnp.float32)
    cos_t = cos * xlen
    phi_t = phi * xlen
    out = cos_t - mask * cos_t * COEF + mask * phi_t * COEF

    m = jnp.max(out, axis=0, keepdims=True)
    lse = m + jnp.log(jnp.sum(jnp.exp(out - m), axis=0, keepdims=True))
    ri = lax.broadcasted_iota(jnp.int32, (B, B), 0)
    ci = lax.broadcasted_iota(jnp.int32, (B, B), 1)
    diag = jnp.sum(jnp.where(ri == ci, out, 0.0), axis=0, keepdims=True)
    logpt = diag - lse
    out_ref[0, 0] = -jnp.mean(logpt)


_tc_loss = pl.pallas_call(
    _tc_loss_body,
    out_shape=jax.ShapeDtypeStruct((1, 1), jnp.float32),
    out_specs=pl.BlockSpec(memory_space=pltpu.SMEM),
)


def kernel(input, target, W):
    w_flat = W.reshape(-1)
    wt = _sc_gather()(w_flat, target).reshape(FEAT, B)
    loss = _tc_loss(input, wt, target.reshape(B, 1), target.reshape(1, B))
    return loss[0, 0]


# speedup vs baseline: 34.7100x; 1.0143x over previous
import functools

import jax
import jax.numpy as jnp
from jax import lax
from jax.experimental import pallas as pl
from jax.experimental.pallas import tpu as pltpu
from jax.experimental.pallas import tpu_sc as plsc

FEAT = 64
B = 1024
C_DIM = 100000
NW = 32
ROWS_W = 16
COLS_W = 128

COEF = 1.0 / (1.0 + max(5.0, 1500.0 / 1.1))

T1 = 0.7071067818211393
T2 = 1.7948965149208059e-09
T3 = -0.7071067792827723
T4 = -1.0


def _sc_gather_body(w_hbm, tgt_hbm, out_hbm, tgt_v, idx_v, rows_v, sem):
    wid = lax.axis_index("s") * 2 + lax.axis_index("c")

    pltpu.sync_copy(tgt_hbm, tgt_v)

    def build(c, carry):
        f = 2 * wid + c // 64
        val = tgt_v[pl.ds((c % 64) * 16, 16)] + f * C_DIM
        idx_v[c // 8, pl.ds((c % 8) * 16, 16)] = val
        return carry

    lax.fori_loop(0, 128, build, 0)

    def fire(j, carry):
        pltpu.async_copy(w_hbm.at[idx_v.at[j]], rows_v.at[j], sem)
        return carry

    lax.fori_loop(0, ROWS_W, fire, 0)

    def drain(j, carry):
        pltpu.make_async_copy(w_hbm.at[idx_v.at[j]], rows_v.at[j], sem).wait()
        return carry

    lax.fori_loop(0, ROWS_W, drain, 0)

    pltpu.sync_copy(rows_v, out_hbm.at[wid])


@functools.cache
def _sc_gather():
    return pl.kernel(
        _sc_gather_body,
        out_type=jax.ShapeDtypeStruct((NW, ROWS_W, COLS_W), jnp.float32),
        mesh=plsc.VectorSubcoreMesh(core_axis_name="c", subcore_axis_name="s"),
        scratch_types=[
            pltpu.VMEM((B,), jnp.int32),
            pltpu.VMEM((ROWS_W, COLS_W), jnp.int32),
            pltpu.VMEM((ROWS_W, COLS_W), jnp.float32),
            pltpu.SemaphoreType.DMA,
        ],
    )


def _tc_loss_body(x_ref, wt_ref, tcol_ref, trow_ref, out_ref):
    x = x_ref[...]
    wt = wt_ref[...]
    tcol = tcol_ref[...]
    trow = trow_ref[...]

    wnorm = jnp.sqrt(jnp.sum(wt * wt, axis=0, keepdims=True))
    scale = jnp.where(wnorm > 1e-5, 1e-5 / jnp.maximum(wnorm, 1e-30), 1.0) * 1e5
    ww = wt * scale
    wlen = jnp.sqrt(jnp.sum(ww * ww, axis=0, keepdims=True))
    wwn = ww / wlen
    xlen = jnp.sqrt(jnp.sum(x * x, axis=1, keepdims=True))
    xn = x / xlen

    cos = lax.dot_general(
        xn, wwn, (((1,), (0,)), ((), ())),
        precision=lax.Precision.HIGHEST,
        preferred_element_type=jnp.float32,
    )

    cos = jnp.clip(cos, -1.0, 1.0)
    cos2 = cos * cos
    cos_m = 8.0 * cos2 * cos2 - 8.0 * cos2 + 1.0
    k = ((cos <= T1).astype(jnp.float32)
         + (cos <= T2).astype(jnp.float32)
         + (cos <= T3).astype(jnp.float32)
         + (cos <= T4).astype(jnp.float32))
    sign = jnp.where(jnp.mod(k, 2.0) == 0.0, 1.0, -1.0)
    phi = sign * cos_m - 2.0 * k

    mask = (tcol == trow).astype(jnp.float32)
    out = xlen * (cos + mask * ((phi - cos) * COEF))

    m = jnp.max(out, axis=0, keepdims=True)
    lse = m + jnp.log(jnp.sum(jnp.exp(out - m), axis=0, keepdims=True))
    ri = lax.broadcasted_iota(jnp.int32, (B, B), 0)
    ci = lax.broadcasted_iota(jnp.int32, (B, B), 1)
    diag = jnp.sum(jnp.where(ri == ci, out, 0.0), axis=0, keepdims=True)
    logpt = diag - lse
    out_ref[0, 0] = -jnp.mean(logpt)


_tc_loss = pl.pallas_call(
    _tc_loss_body,
    out_shape=jax.ShapeDtypeStruct((1, 1), jnp.float32),
    out_specs=pl.BlockSpec(memory_space=pltpu.SMEM),
)


def kernel(input, target, W):
    w_flat = W.reshape(-1)
    wt = _sc_gather()(w_flat, target).reshape(FEAT, B)
    loss = _tc_loss(input, wt, target.reshape(B, 1), target.reshape(1, B))
    return loss[0, 0]
